# baseline (speedup 1.0000x reference)
import functools

import jax
import jax.numpy as jnp
from jax import lax
from jax.experimental import pallas as pl
from jax.experimental.pallas import tpu as pltpu
from jax.experimental.pallas import tpu_sc as plsc

B = 16384
D = 32
NC = 2
NS = 16
NW = NC * NS
L = 16
BPW = B // NW
CH = 128
NCH = BPW // CH


def _mf_kernel(users_hbm, items_hbm, ue_hbm, ie_hbm, ub_hbm, ib_hbm,
               out_hbm, uidx, iidx, ubidx, ibidx, urows, irows, ubuf, ibuf,
               obuf, sem):
    wid = lax.axis_index("s") * NC + lax.axis_index("c")
    base = wid * BPW

    pltpu.sync_copy(users_hbm.at[pl.ds(base, BPW)], uidx)
    pltpu.sync_copy(items_hbm.at[pl.ds(base, BPW)], iidx)

    copies = []
    for j in range(NCH):
        isl = pl.ds(j * CH, CH)
        copies.append(pltpu.async_copy(ue_hbm.at[uidx.at[isl]], urows.at[isl], sem))
        copies.append(pltpu.async_copy(ie_hbm.at[iidx.at[isl]], irows.at[isl], sem))

    def sbody(g, carry):
        sl = pl.ds(g * L, L)
        ubidx[sl] = uidx[sl] >> 4
        ibidx[sl] = iidx[sl] >> 4
        return carry

    lax.fori_loop(0, BPW // L, sbody, 0)

    for j in range(NCH):
        isl = pl.ds(j * CH, CH)
        copies.append(pltpu.async_copy(ub_hbm.at[ubidx.at[isl]], ubuf.at[isl], sem))
        copies.append(pltpu.async_copy(ib_hbm.at[ibidx.at[isl]], ibuf.at[isl], sem))
    for c in copies:
        c.wait()

    lane = lax.iota(jnp.int32, L)

    def gbody(g, carry):
        sl = pl.ds(g * L, L)
        ridx = g * L + lane
        bias = (plsc.load_gather(ubuf, [ridx, uidx[sl] & 15]) +
                plsc.load_gather(ibuf, [ridx, iidx[sl] & 15]))
        acc = jnp.zeros((L,), jnp.float32)
        for r in range(L):
            row = g * L + r
            u0 = urows[row, pl.ds(0, L)]
            u1 = urows[row, pl.ds(L, L)]
            i0 = irows[row, pl.ds(0, L)]
            i1 = irows[row, pl.ds(L, L)]
            tot = jnp.sum(u0 * i0 + u1 * i1)
            acc = jnp.where(lane == r, jnp.full((L,), tot), acc)
        obuf[sl] = acc + bias
        return carry

    lax.fori_loop(0, BPW // L, gbody, 0)

    pltpu.sync_copy(obuf, out_hbm.at[pl.ds(base, BPW)])


@jax.jit
def kernel(vector, user_emb, item_emb, user_bias, item_bias):
    users = vector[0]
    items = vector[1]
    ub16 = user_bias.reshape(user_bias.shape[0] // L, L)
    ib16 = item_bias.reshape(item_bias.shape[0] // L, L)
    run = functools.partial(
        pl.kernel,
        out_type=jax.ShapeDtypeStruct((B,), jnp.float32),
        mesh=plsc.VectorSubcoreMesh(core_axis_name="c", subcore_axis_name="s"),
        compiler_params=pltpu.CompilerParams(
            needs_layout_passes=False, use_tc_tiling_on_sc=False),
        scratch_types=[
            pltpu.VMEM((BPW,), jnp.int32),
            pltpu.VMEM((BPW,), jnp.int32),
            pltpu.VMEM((BPW,), jnp.int32),
            pltpu.VMEM((BPW,), jnp.int32),
            pltpu.VMEM((BPW, D), jnp.float32),
            pltpu.VMEM((BPW, D), jnp.float32),
            pltpu.VMEM((BPW, L), jnp.float32),
            pltpu.VMEM((BPW, L), jnp.float32),
            pltpu.VMEM((BPW,), jnp.float32),
            pltpu.SemaphoreType.DMA,
        ],
    )(_mf_kernel)
    return run(users, items, user_emb, item_emb, ub16, ib16)


# speedup vs baseline: 2.6458x; 2.6458x over previous
import functools

import jax
import jax.numpy as jnp
from jax import lax
from jax.experimental import pallas as pl
from jax.experimental.pallas import tpu as pltpu
from jax.experimental.pallas import tpu_sc as plsc

B = 16384
D = 32
NC = 2
NS = 16
NW = NC * NS
L = 16
BPW = B // NW
CHKW = 4
NCHK = BPW // CHKW


def _mf_kernel(users_hbm, items_hbm, ueT_hbm, ieT_hbm, ub_hbm, ib_hbm,
               out_hbm, uidx, iidx, uwin, iwin, ubw, ibw, obuf, sem0, sem1):
    wid = lax.axis_index("s") * NC + lax.axis_index("c")
    base = wid * BPW

    pltpu.sync_copy(users_hbm.at[pl.ds(base, BPW)], uidx.at[pl.ds(0, BPW)])
    pltpu.sync_copy(items_hbm.at[pl.ds(base, BPW)], iidx.at[pl.ds(0, BPW)])

    lane = lax.iota(jnp.int32, L)
    sems = [sem0, sem1]

    def fire(c, buf):
        sem = sems[buf]
        iv_u = uidx[pl.ds(c * CHKW, L)]
        iv_i = iidx[pl.ds(c * CHKW, L)]
        for e in range(CHKW):
            u = iv_u[e]
            v = iv_i[e]
            wu = pl.multiple_of((u >> 7) * 128, 128)
            wv = pl.multiple_of((v >> 7) * 128, 128)
            pltpu.async_copy(ueT_hbm.at[:, pl.ds(wu, 128)], uwin.at[buf, e], sem)
            pltpu.async_copy(ieT_hbm.at[:, pl.ds(wv, 128)], iwin.at[buf, e], sem)
            pltpu.async_copy(ub_hbm.at[pl.ds(wu, 128)], ubw.at[buf, e], sem)
            pltpu.async_copy(ib_hbm.at[pl.ds(wv, 128)], ibw.at[buf, e], sem)

    def drain(buf):
        sem = sems[buf]
        for e in range(CHKW):
            pltpu.make_async_copy(ueT_hbm.at[:, pl.ds(0, 128)], uwin.at[buf, e], sem).wait()
            pltpu.make_async_copy(ieT_hbm.at[:, pl.ds(0, 128)], iwin.at[buf, e], sem).wait()
            pltpu.make_async_copy(ub_hbm.at[pl.ds(0, 128)], ubw.at[buf, e], sem).wait()
            pltpu.make_async_copy(ib_hbm.at[pl.ds(0, 128)], ibw.at[buf, e], sem).wait()

    def compute(c, buf, acc):
        iv_u = uidx[pl.ds(c * CHKW, L)]
        iv_i = iidx[pl.ds(c * CHKW, L)]
        bsp = jnp.full((L,), buf, jnp.int32)
        for e in range(CHKW):
            cu = jnp.full((L,), iv_u[e] & 127, jnp.int32)
            cv = jnp.full((L,), iv_i[e] & 127, jnp.int32)
            esp = jnp.full((L,), e, jnp.int32)
            du0 = plsc.load_gather(uwin, [bsp, esp, lane, cu])
            du1 = plsc.load_gather(uwin, [bsp, esp, lane + L, cu])
            di0 = plsc.load_gather(iwin, [bsp, esp, lane, cv])
            di1 = plsc.load_gather(iwin, [bsp, esp, lane + L, cv])
            tot = jnp.sum(du0 * di0 + du1 * di1)
            bu = plsc.load_gather(ubw, [bsp, esp, cu])
            bi = plsc.load_gather(ibw, [bsp, esp, cv])
            res = jnp.full((L,), tot) + bu + bi
            lanepos = (c & 3) * CHKW + e
            acc = jnp.where(lane == lanepos, res, acc)
        return acc

    def store_maybe(c, acc):
        @pl.when((c & 3) == 3)
        def _():
            obuf[pl.ds((c >> 2) * L, L)] = acc
        return jnp.where((c & 3) == 3, jnp.zeros((L,), jnp.float32), acc)

    fire(0, 0)

    def body(tp, acc):
        ca = 2 * tp
        cb = 2 * tp + 1
        fire(cb, 1)
        drain(0)
        acc = compute(ca, 0, acc)
        acc = store_maybe(ca, acc)

        @pl.when(tp < NCHK // 2 - 1)
        def _():
            fire(ca + 2, 0)

        drain(1)
        acc = compute(cb, 1, acc)
        acc = store_maybe(cb, acc)
        return acc

    lax.fori_loop(0, NCHK // 2, body, jnp.zeros((L,), jnp.float32))

    pltpu.sync_copy(obuf, out_hbm.at[pl.ds(base, BPW)])


@jax.jit
def kernel(vector, user_emb, item_emb, user_bias, item_bias):
    users = vector[0]
    items = vector[1]
    ueT = user_emb.T
    ieT = item_emb.T
    ub1 = user_bias[:, 0]
    ib1 = item_bias[:, 0]
    run = functools.partial(
        pl.kernel,
        out_type=jax.ShapeDtypeStruct((B,), jnp.float32),
        mesh=plsc.VectorSubcoreMesh(core_axis_name="c", subcore_axis_name="s"),
        compiler_params=pltpu.CompilerParams(
            needs_layout_passes=False, use_tc_tiling_on_sc=True),
        scratch_types=[
            pltpu.VMEM((BPW + L,), jnp.int32),
            pltpu.VMEM((BPW + L,), jnp.int32),
            pltpu.VMEM((2, CHKW, D, 128), jnp.float32),
            pltpu.VMEM((2, CHKW, D, 128), jnp.float32),
            pltpu.VMEM((2, CHKW, 128), jnp.float32),
            pltpu.VMEM((2, CHKW, 128), jnp.float32),
            pltpu.VMEM((BPW,), jnp.float32),
            pltpu.SemaphoreType.DMA,
            pltpu.SemaphoreType.DMA,
        ],
    )(_mf_kernel)
    return run(users, items, ueT, ieT, ub1, ib1)
